# baseline (speedup 1.0000x reference)
import functools

import jax
import jax.numpy as jnp
from jax import lax
from jax.experimental import pallas as pl
from jax.experimental.pallas import tpu as pltpu
from jax.experimental.pallas import tpu_sc as plsc

N_NODES = 10000
N_EDGES = 320000
D_FEAT = 128
D_HID = 16
N_CLS = 7
F = 16

NC = 2
NS = 16
NW = NC * NS
NPAD = 10240
ROWS_PER_TILE = NPAD // NS
CHUNK = 128
NCHUNKS = N_EDGES // CHUNK
CHUNKS_BASE = NCHUNKS // NW
CHUNKS_EXTRA = NCHUNKS - CHUNKS_BASE * NW
RING = 8
NGROUPS = CHUNKS_BASE // RING
NEPI = CHUNKS_BASE - RING * NGROUPS
DEG_BASE = NCHUNKS // NS
DEG_EXTRA = NCHUNKS - DEG_BASE * NS
DRING = 8
DEG_GROUPS = DEG_BASE // DRING
DEG_EPI = DEG_BASE - DRING * DEG_GROUPS

_MESH = plsc.VectorSubcoreMesh(
    core_axis_name="c", subcore_axis_name="s", num_cores=NC, num_subcores=NS
)
_SC_PARAMS = pltpu.CompilerParams(use_tc_tiling_on_sc=False, needs_layout_passes=False)


def _bcast(ref1d, i):
    return plsc.load_gather(ref1d, [jnp.full((16,), i, jnp.int32)])


def _bcast2(ref2d, i, k):
    return plsc.load_gather(
        ref2d, [jnp.full((16,), i, jnp.int32), jnp.full((16,), k, jnp.int32)]
    )


def _rsqrt_newton(d):
    i = plsc.bitcast(d, jnp.int32)
    y = plsc.bitcast(jnp.int32(0x5F3759DF) - (i >> 1), jnp.float32)
    for _ in range(3):
        y = y * (1.5 - 0.5 * d * y * y)
    return y


def _prop_pipeline(u_sh, acc_sh, sidx2, didx2, sidx_x, didx_x, msg, gsem, ssem,
                   src2_hbm, dst2_hbm, wid):
    def _gather(j, b):
        return pltpu.async_copy(u_sh.at[sidx2.at[j]], msg[b], gsem[b])

    def _gather_wait(j, b):
        pltpu.make_async_copy(u_sh.at[sidx2.at[j]], msg[b], gsem[b]).wait()

    def _scatter(j, b):
        return pltpu.async_copy(msg[b], acc_sh.at[didx2.at[j]], ssem[b], add=True)

    def _scatter_wait(j, b):
        pltpu.make_async_copy(msg[b], acc_sh.at[didx2.at[j]], ssem[b]).wait()

    for b in range(RING - 1):
        _gather(b, b)

    def grp(g, carry):
        for b in range(RING):
            j = RING * g + b
            _gather_wait(j, b)
            _scatter(j, b)
            bp = (b + RING - 1) % RING

            if b == 0:
                @pl.when(g > 0)
                def _():
                    _scatter_wait(j - 1, bp)
            else:
                _scatter_wait(j - 1, bp)

            if b <= NEPI:
                _gather(j + RING - 1, bp)
            else:
                @pl.when(j + RING - 1 < CHUNKS_BASE)
                def _():
                    _gather(j + RING - 1, bp)

        return carry

    lax.fori_loop(0, NGROUPS, grp, 0)

    for k in range(NEPI):
        j = RING * NGROUPS + k
        b = j % RING
        _gather_wait(j, b)
        pltpu.sync_copy(msg[b], acc_sh.at[didx2.at[j]], add=True)

    _scatter_wait(RING * NGROUPS - 1, (RING * NGROUPS - 1) % RING)

    @pl.when(wid < CHUNKS_EXTRA)
    def _():
        pltpu.sync_copy(src2_hbm.at[NW * CHUNKS_BASE + wid], sidx_x)
        pltpu.sync_copy(dst2_hbm.at[NW * CHUNKS_BASE + wid], didx_x)
        pltpu.async_copy(u_sh.at[sidx_x], msg[0], gsem[0]).wait()
        pltpu.sync_copy(msg[0], acc_sh.at[didx_x], add=True)


@functools.partial(
    pl.kernel,
    out_type=[
        jax.ShapeDtypeStruct((NC, NPAD, F), jnp.float32),
        jax.ShapeDtypeStruct((NPAD, F), jnp.float32),
        jax.ShapeDtypeStruct((NPAD,), jnp.float32),
    ],
    mesh=_MESH,
    scratch_types=[
        pltpu.VMEM((DEG_BASE, CHUNK), jnp.int32),
        pltpu.VMEM((CHUNKS_BASE, CHUNK), jnp.int32),
        pltpu.VMEM((CHUNKS_BASE, CHUNK), jnp.int32),
        pltpu.VMEM((CHUNK,), jnp.int32),
        pltpu.VMEM((CHUNK,), jnp.int32),
        [pltpu.VMEM((CHUNK, F), jnp.float32) for _ in range(RING)],
        pltpu.VMEM((ROWS_PER_TILE, F), jnp.float32),
        pltpu.VMEM((ROWS_PER_TILE,), jnp.float32),
        pltpu.VMEM((CHUNK,), jnp.float32),
        pltpu.VMEM((CHUNK,), jnp.float32),
        pltpu.VMEM((CHUNK, F), jnp.float32),
        pltpu.VMEM_SHARED((NPAD, F), jnp.float32),
        pltpu.VMEM_SHARED((NPAD,), jnp.float32),
        pltpu.VMEM_SHARED((NPAD, F), jnp.float32),
        pltpu.SemaphoreType.DMA,
        [pltpu.SemaphoreType.DMA for _ in range(RING)],
        [pltpu.SemaphoreType.DMA for _ in range(RING)],
    ],
    compiler_params=_SC_PARAMS,
)
def _mega1(
    z1_hbm, src2_hbm, dst2_hbm, s1_hbm, u1_hbm, dis_hbm,
    didx_deg, sidx2, didx2, sidx_x, didx_x, msg, zloc, dbuf, ones_v, zero1,
    zero_v, acc_sh, deg_sh, u_sh, isem, gsem, ssem,
):
    c = lax.axis_index("c")
    s = lax.axis_index("s")
    wid = c * NS + s
    r0 = s * ROWS_PER_TILE
    base_d = DEG_BASE * s + jnp.minimum(s, DEG_EXTRA)

    pltpu.async_copy(dst2_hbm.at[pl.ds(base_d, DEG_BASE)], didx_deg, isem)
    pltpu.async_copy(src2_hbm.at[pl.ds(wid * CHUNKS_BASE, CHUNKS_BASE)], sidx2, isem)
    pltpu.async_copy(dst2_hbm.at[pl.ds(wid * CHUNKS_BASE, CHUNKS_BASE)], didx2, isem)
    pltpu.async_copy(z1_hbm.at[pl.ds(r0, ROWS_PER_TILE)], zloc, isem)

    for k in range(CHUNK // 16):
        ones_v[pl.ds(k * 16, 16)] = jnp.ones((16,), jnp.float32)
        zero1[pl.ds(k * 16, 16)] = jnp.zeros((16,), jnp.float32)

    def zr(r, carry):
        zero_v[r, :] = jnp.zeros((F,), jnp.float32)
        return carry

    lax.fori_loop(0, CHUNK, zr, 0)

    def zb(k, carry):
        pltpu.sync_copy(zero_v, acc_sh.at[pl.ds(r0 + k * CHUNK, CHUNK)])
        pltpu.sync_copy(zero1, deg_sh.at[pl.ds(r0 + k * CHUNK, CHUNK)])
        return carry

    lax.fori_loop(0, ROWS_PER_TILE // CHUNK, zb, 0)

    pltpu.make_async_copy(dst2_hbm.at[pl.ds(base_d, DEG_BASE)], didx_deg, isem).wait()
    pltpu.make_async_copy(
        src2_hbm.at[pl.ds(wid * CHUNKS_BASE, CHUNKS_BASE)], sidx2, isem
    ).wait()
    pltpu.make_async_copy(
        dst2_hbm.at[pl.ds(wid * CHUNKS_BASE, CHUNKS_BASE)], didx2, isem
    ).wait()
    pltpu.make_async_copy(z1_hbm.at[pl.ds(r0, ROWS_PER_TILE)], zloc, isem).wait()
    plsc.subcore_barrier()

    def dgrp(g, carry):
        for b in range(DRING):
            j = DRING * g + b

            @pl.when(g > 0)
            def _():
                pltpu.make_async_copy(
                    ones_v, deg_sh.at[didx_deg.at[j - DRING]], ssem[b]
                ).wait()

            pltpu.async_copy(ones_v, deg_sh.at[didx_deg.at[j]], ssem[b], add=True)
        return carry

    lax.fori_loop(0, DEG_GROUPS, dgrp, 0)
    for k in range(DEG_EPI):
        pltpu.sync_copy(ones_v, deg_sh.at[didx_deg.at[DRING * DEG_GROUPS + k]], add=True)
    for b in range(DRING):
        pltpu.make_async_copy(
            ones_v, deg_sh.at[didx_deg.at[DRING * (DEG_GROUPS - 1) + b]], ssem[b]
        ).wait()

    @pl.when(s < DEG_EXTRA)
    def _():
        pltpu.sync_copy(dst2_hbm.at[base_d + DEG_BASE], didx_x)
        pltpu.sync_copy(ones_v, deg_sh.at[didx_x], add=True)

    plsc.subcore_barrier()

    pltpu.sync_copy(deg_sh.at[pl.ds(r0, ROWS_PER_TILE)], dbuf)
    for v in range(ROWS_PER_TILE // 16):
        d = dbuf[pl.ds(v * 16, 16)] + 1.0
        dbuf[pl.ds(v * 16, 16)] = _rsqrt_newton(d)

    def urow(i, carry):
        zloc[i, :] = zloc[i, :] * _bcast(dbuf, i)
        return carry

    lax.fori_loop(0, ROWS_PER_TILE, urow, 0)
    pltpu.sync_copy(zloc, u_sh.at[pl.ds(r0, ROWS_PER_TILE)])

    @pl.when(c == 0)
    def _():
        pltpu.async_copy(zloc, u1_hbm.at[pl.ds(r0, ROWS_PER_TILE)], isem)
        pltpu.async_copy(dbuf, dis_hbm.at[pl.ds(r0, ROWS_PER_TILE)], isem)

    plsc.subcore_barrier()

    _prop_pipeline(u_sh, acc_sh, sidx2, didx2, sidx_x, didx_x, msg, gsem, ssem,
                   src2_hbm, dst2_hbm, wid)
    plsc.subcore_barrier()

    pltpu.sync_copy(
        acc_sh.at[pl.ds(r0, ROWS_PER_TILE)], s1_hbm.at[c, pl.ds(r0, ROWS_PER_TILE)]
    )

    @pl.when(c == 0)
    def _():
        pltpu.make_async_copy(zloc, u1_hbm.at[pl.ds(r0, ROWS_PER_TILE)], isem).wait()
        pltpu.make_async_copy(dbuf, dis_hbm.at[pl.ds(r0, ROWS_PER_TILE)], isem).wait()


@functools.partial(
    pl.kernel,
    out_type=[
        jax.ShapeDtypeStruct((NC, NPAD, F), jnp.float32),
        jax.ShapeDtypeStruct((NPAD, F), jnp.float32),
    ],
    mesh=_MESH,
    scratch_types=[
        pltpu.VMEM((CHUNKS_BASE, CHUNK), jnp.int32),
        pltpu.VMEM((CHUNKS_BASE, CHUNK), jnp.int32),
        pltpu.VMEM((CHUNK,), jnp.int32),
        pltpu.VMEM((CHUNK,), jnp.int32),
        [pltpu.VMEM((CHUNK, F), jnp.float32) for _ in range(RING)],
        pltpu.VMEM((ROWS_PER_TILE, F), jnp.float32),
        pltpu.VMEM((ROWS_PER_TILE, F), jnp.float32),
        pltpu.VMEM((ROWS_PER_TILE, F), jnp.float32),
        pltpu.VMEM((ROWS_PER_TILE, F), jnp.float32),
        pltpu.VMEM((ROWS_PER_TILE,), jnp.float32),
        pltpu.VMEM((16,), jnp.float32),
        pltpu.VMEM((16, 16), jnp.float32),
        pltpu.VMEM((CHUNK, F), jnp.float32),
        pltpu.VMEM_SHARED((NPAD, F), jnp.float32),
        pltpu.VMEM_SHARED((NPAD, F), jnp.float32),
        pltpu.SemaphoreType.DMA,
        [pltpu.SemaphoreType.DMA for _ in range(RING)],
        [pltpu.SemaphoreType.DMA for _ in range(RING)],
    ],
    compiler_params=_SC_PARAMS,
)
def _mega2(
    s1_hbm, u1_hbm, dis_hbm, b1_hbm, w2p_hbm, src2_hbm, dst2_hbm,
    s2_hbm, u2_hbm,
    sidx2, didx2, sidx_x, didx_x, msg, s0loc, s1loc, u1loc, hloc, dbuf,
    b1loc, w2loc, zero_v, acc_sh, u_sh, isem, gsem, ssem,
):
    c = lax.axis_index("c")
    s = lax.axis_index("s")
    wid = c * NS + s
    r0 = s * ROWS_PER_TILE

    pltpu.async_copy(src2_hbm.at[pl.ds(wid * CHUNKS_BASE, CHUNKS_BASE)], sidx2, isem)
    pltpu.async_copy(dst2_hbm.at[pl.ds(wid * CHUNKS_BASE, CHUNKS_BASE)], didx2, isem)
    pltpu.async_copy(s1_hbm.at[0, pl.ds(r0, ROWS_PER_TILE)], s0loc, isem)
    pltpu.async_copy(s1_hbm.at[1, pl.ds(r0, ROWS_PER_TILE)], s1loc, isem)
    pltpu.async_copy(u1_hbm.at[pl.ds(r0, ROWS_PER_TILE)], u1loc, isem)
    pltpu.async_copy(dis_hbm.at[pl.ds(r0, ROWS_PER_TILE)], dbuf, isem)
    pltpu.async_copy(b1_hbm, b1loc, isem)
    pltpu.async_copy(w2p_hbm, w2loc, isem)

    def zr(r, carry):
        zero_v[r, :] = jnp.zeros((F,), jnp.float32)
        return carry

    lax.fori_loop(0, CHUNK, zr, 0)

    def zb(k, carry):
        pltpu.sync_copy(zero_v, acc_sh.at[pl.ds(r0 + k * CHUNK, CHUNK)])
        return carry

    lax.fori_loop(0, ROWS_PER_TILE // CHUNK, zb, 0)

    pltpu.make_async_copy(
        src2_hbm.at[pl.ds(wid * CHUNKS_BASE, CHUNKS_BASE)], sidx2, isem
    ).wait()
    pltpu.make_async_copy(
        dst2_hbm.at[pl.ds(wid * CHUNKS_BASE, CHUNKS_BASE)], didx2, isem
    ).wait()
    pltpu.make_async_copy(s1_hbm.at[0, pl.ds(r0, ROWS_PER_TILE)], s0loc, isem).wait()
    pltpu.make_async_copy(s1_hbm.at[1, pl.ds(r0, ROWS_PER_TILE)], s1loc, isem).wait()
    pltpu.make_async_copy(u1_hbm.at[pl.ds(r0, ROWS_PER_TILE)], u1loc, isem).wait()
    pltpu.make_async_copy(dis_hbm.at[pl.ds(r0, ROWS_PER_TILE)], dbuf, isem).wait()
    pltpu.make_async_copy(b1_hbm, b1loc, isem).wait()
    pltpu.make_async_copy(w2p_hbm, w2loc, isem).wait()

    b1v = b1loc[...]

    def hrow(i, carry):
        row = (s0loc[i, :] + s1loc[i, :] + u1loc[i, :]) * _bcast(dbuf, i) + b1v
        hloc[i, :] = jnp.maximum(row, 0.0)
        return carry

    lax.fori_loop(0, ROWS_PER_TILE, hrow, 0)

    w = [w2loc[k, :] for k in range(16)]

    def mrow(i, carry):
        t = [w[k] * _bcast2(hloc, i, k) for k in range(16)]
        while len(t) > 1:
            t = [t[p] + t[p + 1] for p in range(0, len(t), 2)]
        s0loc[i, :] = t[0] * _bcast(dbuf, i)
        return carry

    lax.fori_loop(0, ROWS_PER_TILE, mrow, 0)

    pltpu.sync_copy(s0loc, u_sh.at[pl.ds(r0, ROWS_PER_TILE)])

    @pl.when(c == 0)
    def _():
        pltpu.async_copy(s0loc, u2_hbm.at[pl.ds(r0, ROWS_PER_TILE)], isem)

    plsc.subcore_barrier()

    _prop_pipeline(u_sh, acc_sh, sidx2, didx2, sidx_x, didx_x, msg, gsem, ssem,
                   src2_hbm, dst2_hbm, wid)
    plsc.subcore_barrier()

    pltpu.sync_copy(
        acc_sh.at[pl.ds(r0, ROWS_PER_TILE)], s2_hbm.at[c, pl.ds(r0, ROWS_PER_TILE)]
    )

    @pl.when(c == 0)
    def _():
        pltpu.make_async_copy(s0loc, u2_hbm.at[pl.ds(r0, ROWS_PER_TILE)], isem).wait()


def _tc0_body(x_ref, w1_ref, z1_ref):
    z = jnp.dot(x_ref[...], w1_ref[...], preferred_element_type=jnp.float32)
    z1_ref[pl.ds(0, N_NODES), :] = z
    z1_ref[pl.ds(N_NODES, NPAD - N_NODES), :] = jnp.zeros(
        (NPAD - N_NODES, F), jnp.float32
    )


def _tc3_body(dis_ref, u2_ref, s2_ref, b2p_ref, out_ref):
    dis = dis_ref[pl.ds(0, N_NODES)]
    ssum = s2_ref[0, :N_NODES, :] + s2_ref[1, :N_NODES, :]
    y = (ssum + u2_ref[:N_NODES, :]) * dis[:, None] + b2p_ref[...][None, :]
    col = lax.broadcasted_iota(jnp.int32, (N_NODES, F), 1)
    y = jnp.where(col < N_CLS, y, -1e30)
    m = jnp.max(y, axis=1, keepdims=True)
    lse = jnp.log(jnp.sum(jnp.exp(y - m), axis=1, keepdims=True))
    ls = y - m - lse
    out_ref[...] = ls[:, :N_CLS]


_tc0 = pl.pallas_call(
    _tc0_body,
    out_shape=jax.ShapeDtypeStruct((NPAD, F), jnp.float32),
)

_tc3 = pl.pallas_call(
    _tc3_body,
    out_shape=jax.ShapeDtypeStruct((N_NODES, N_CLS), jnp.float32),
)


def kernel(x, edge_index, W1, b1, W2, b2):
    src2 = edge_index[0].astype(jnp.int32).reshape(NCHUNKS, CHUNK)
    dst2 = edge_index[1].astype(jnp.int32).reshape(NCHUNKS, CHUNK)
    w2p = jnp.pad(W2, ((0, 0), (0, F - N_CLS)))
    b2p = jnp.pad(b2, (0, F - N_CLS))

    z1 = _tc0(x, W1)
    s1, u1, dis = _mega1(z1, src2, dst2)
    s2, u2 = _mega2(s1, u1, dis, b1, w2p, src2, dst2)
    return _tc3(dis, u2, s2, b2p)


# speedup vs baseline: 1.0656x; 1.0656x over previous
import functools

import jax
import jax.numpy as jnp
from jax import lax
from jax.experimental import pallas as pl
from jax.experimental.pallas import tpu as pltpu
from jax.experimental.pallas import tpu_sc as plsc

N_NODES = 10000
N_EDGES = 320000
D_FEAT = 128
D_HID = 16
N_CLS = 7
F = 16

NC = 2
NS = 16
NW = NC * NS
NPAD = 10240
ROWS_PER_TILE = NPAD // NS
CHUNK = 128
NCHUNKS = N_EDGES // CHUNK
CHUNKS_BASE = NCHUNKS // NW
CHUNKS_EXTRA = NCHUNKS - CHUNKS_BASE * NW
RING = 8
NGROUPS = CHUNKS_BASE // RING
NEPI = CHUNKS_BASE - RING * NGROUPS

_MESH = plsc.VectorSubcoreMesh(
    core_axis_name="c", subcore_axis_name="s", num_cores=NC, num_subcores=NS
)


@functools.partial(
    pl.kernel,
    out_type=jax.ShapeDtypeStruct((NC, NPAD), jnp.float32),
    mesh=_MESH,
    scratch_types=[
        pltpu.VMEM((CHUNKS_BASE, CHUNK), jnp.int32),
        pltpu.VMEM((CHUNK,), jnp.int32),
        pltpu.VMEM((CHUNK,), jnp.float32),
        pltpu.VMEM((80,), jnp.float32),
        pltpu.VMEM_SHARED((NPAD,), jnp.float32),
        pltpu.SemaphoreType.DMA,
        [pltpu.SemaphoreType.DMA for _ in range(4)],
    ],
    compiler_params=pltpu.CompilerParams(use_tc_tiling_on_sc=False),
)
def _deg_kernel(dst2_hbm, degp_hbm, didx2, idx_x, ones_v, zero_v, deg_sh, sem, dsem):
    c = lax.axis_index("c")
    s = lax.axis_index("s")
    wid = c * NS + s

    pltpu.async_copy(dst2_hbm.at[pl.ds(wid * CHUNKS_BASE, CHUNKS_BASE)], didx2, sem)

    for k in range(CHUNK // 16):
        ones_v[pl.ds(k * 16, 16)] = jnp.ones((16,), jnp.float32)
    for k in range(80 // 16):
        zero_v[pl.ds(k * 16, 16)] = jnp.zeros((16,), jnp.float32)

    def zb(k, carry):
        pltpu.sync_copy(zero_v, deg_sh.at[pl.ds(s * ROWS_PER_TILE + k * 80, 80)])
        return carry

    lax.fori_loop(0, ROWS_PER_TILE // 80, zb, 0)
    pltpu.make_async_copy(
        dst2_hbm.at[pl.ds(wid * CHUNKS_BASE, CHUNKS_BASE)], didx2, sem
    ).wait()
    plsc.subcore_barrier()

    def body(g, carry):
        for b in range(4):
            j = 4 * g + b

            @pl.when(g > 0)
            def _():
                pltpu.make_async_copy(
                    ones_v, deg_sh.at[didx2.at[j - 4]], dsem[b]
                ).wait()

            pltpu.async_copy(ones_v, deg_sh.at[didx2.at[j]], dsem[b], add=True)
        return carry

    lax.fori_loop(0, CHUNKS_BASE // 4, body, 0)
    for k in range(CHUNKS_BASE - 4 * (CHUNKS_BASE // 4)):
        pltpu.sync_copy(ones_v, deg_sh.at[didx2.at[4 * (CHUNKS_BASE // 4) + k]], add=True)
    for b in range(4):
        pltpu.make_async_copy(
            ones_v, deg_sh.at[didx2.at[4 * ((CHUNKS_BASE // 4) - 1) + b]], dsem[b]
        ).wait()

    @pl.when(wid < CHUNKS_EXTRA)
    def _():
        pltpu.sync_copy(dst2_hbm.at[NW * CHUNKS_BASE + wid], idx_x)
        pltpu.sync_copy(ones_v, deg_sh.at[idx_x], add=True)

    plsc.subcore_barrier()

    r0 = s * ROWS_PER_TILE
    pltpu.sync_copy(
        deg_sh.at[pl.ds(r0, ROWS_PER_TILE)], degp_hbm.at[c, pl.ds(r0, ROWS_PER_TILE)]
    )


@functools.partial(
    pl.kernel,
    out_type=jax.ShapeDtypeStruct((NC, NPAD, F), jnp.float32),
    mesh=_MESH,
    scratch_types=[
        pltpu.VMEM((CHUNKS_BASE, CHUNK), jnp.int32),
        pltpu.VMEM((CHUNKS_BASE, CHUNK), jnp.int32),
        pltpu.VMEM((CHUNK,), jnp.int32),
        pltpu.VMEM((CHUNK,), jnp.int32),
        [pltpu.VMEM((CHUNK, F), jnp.float32) for _ in range(RING)],
        pltpu.VMEM((128, F), jnp.float32),
        pltpu.VMEM_SHARED((NPAD, F), jnp.float32),
        pltpu.VMEM_SHARED((NPAD, F), jnp.float32),
        pltpu.SemaphoreType.DMA,
        [pltpu.SemaphoreType.DMA for _ in range(RING)],
        [pltpu.SemaphoreType.DMA for _ in range(RING)],
    ],
    compiler_params=pltpu.CompilerParams(use_tc_tiling_on_sc=False),
)
def _prop_kernel(
    u_hbm, src2_hbm, dst2_hbm, outp_hbm,
    sidx2, didx2, sidx_x, didx_x, msg, zero_v, acc_sh, u_sh, isem, gsem, ssem,
):
    c = lax.axis_index("c")
    s = lax.axis_index("s")
    wid = c * NS + s

    pltpu.async_copy(src2_hbm.at[pl.ds(wid * CHUNKS_BASE, CHUNKS_BASE)], sidx2, isem)
    pltpu.async_copy(dst2_hbm.at[pl.ds(wid * CHUNKS_BASE, CHUNKS_BASE)], didx2, isem)

    @pl.when(s < NS - 1)
    def _():
        pltpu.async_copy(
            u_hbm.at[pl.ds(s * ROWS_PER_TILE, ROWS_PER_TILE)],
            u_sh.at[pl.ds(s * ROWS_PER_TILE, ROWS_PER_TILE)],
            isem,
        )

    @pl.when(s == NS - 1)
    def _():
        pltpu.async_copy(
            u_hbm.at[pl.ds((NS - 1) * ROWS_PER_TILE, N_NODES - (NS - 1) * ROWS_PER_TILE)],
            u_sh.at[pl.ds((NS - 1) * ROWS_PER_TILE, N_NODES - (NS - 1) * ROWS_PER_TILE)],
            isem,
        )

    def zr(r, carry):
        zero_v[r, :] = jnp.zeros((F,), jnp.float32)
        return carry

    lax.fori_loop(0, 128, zr, 0)

    def zb(k, carry):
        pltpu.sync_copy(zero_v, acc_sh.at[pl.ds(s * ROWS_PER_TILE + k * 128, 128)])
        return carry

    lax.fori_loop(0, ROWS_PER_TILE // 128, zb, 0)
    pltpu.make_async_copy(
        src2_hbm.at[pl.ds(wid * CHUNKS_BASE, CHUNKS_BASE)], sidx2, isem
    ).wait()
    pltpu.make_async_copy(
        dst2_hbm.at[pl.ds(wid * CHUNKS_BASE, CHUNKS_BASE)], didx2, isem
    ).wait()

    @pl.when(s < NS - 1)
    def _():
        pltpu.make_async_copy(
            u_hbm.at[pl.ds(s * ROWS_PER_TILE, ROWS_PER_TILE)],
            u_sh.at[pl.ds(s * ROWS_PER_TILE, ROWS_PER_TILE)],
            isem,
        ).wait()

    @pl.when(s == NS - 1)
    def _():
        pltpu.make_async_copy(
            u_hbm.at[pl.ds((NS - 1) * ROWS_PER_TILE, N_NODES - (NS - 1) * ROWS_PER_TILE)],
            u_sh.at[pl.ds((NS - 1) * ROWS_PER_TILE, N_NODES - (NS - 1) * ROWS_PER_TILE)],
            isem,
        ).wait()

    plsc.subcore_barrier()

    def _gather(j, b):
        return pltpu.async_copy(u_sh.at[sidx2.at[j]], msg[b], gsem[b])

    def _gather_wait(j, b):
        pltpu.make_async_copy(u_sh.at[sidx2.at[j]], msg[b], gsem[b]).wait()

    def _scatter(j, b):
        return pltpu.async_copy(msg[b], acc_sh.at[didx2.at[j]], ssem[b], add=True)

    def _scatter_wait(j, b):
        pltpu.make_async_copy(msg[b], acc_sh.at[didx2.at[j]], ssem[b]).wait()

    for b in range(RING - 1):
        _gather(b, b)

    def grp(g, carry):
        for b in range(RING):
            j = RING * g + b
            _gather_wait(j, b)
            _scatter(j, b)
            bp = (b + RING - 1) % RING

            if b == 0:
                @pl.when(g > 0)
                def _():
                    _scatter_wait(j - 1, bp)
            else:
                _scatter_wait(j - 1, bp)

            if b <= NEPI:
                _gather(j + RING - 1, bp)
            else:
                @pl.when(j + RING - 1 < CHUNKS_BASE)
                def _():
                    _gather(j + RING - 1, bp)

        return carry

    lax.fori_loop(0, NGROUPS, grp, 0)

    for k in range(NEPI):
        j = RING * NGROUPS + k
        b = j % RING
        _gather_wait(j, b)
        pltpu.sync_copy(msg[b], acc_sh.at[didx2.at[j]], add=True)

    _scatter_wait(RING * NGROUPS - 1, (RING * NGROUPS - 1) % RING)

    @pl.when(wid < CHUNKS_EXTRA)
    def _():
        pltpu.sync_copy(src2_hbm.at[NW * CHUNKS_BASE + wid], sidx_x)
        pltpu.sync_copy(dst2_hbm.at[NW * CHUNKS_BASE + wid], didx_x)
        pltpu.async_copy(u_sh.at[sidx_x], msg[0], gsem[0]).wait()
        pltpu.sync_copy(msg[0], acc_sh.at[didx_x], add=True)

    plsc.subcore_barrier()

    r0 = s * ROWS_PER_TILE
    pltpu.sync_copy(
        acc_sh.at[pl.ds(r0, ROWS_PER_TILE)], outp_hbm.at[c, pl.ds(r0, ROWS_PER_TILE)]
    )


def _tc1_body(x_ref, w1_ref, degp_ref, dis_ref, u1_ref):
    deg = 1.0 + degp_ref[0, :N_NODES] + degp_ref[1, :N_NODES]
    dis = lax.rsqrt(deg)
    z = jnp.dot(x_ref[...], w1_ref[...], preferred_element_type=jnp.float32)
    dis_ref[...] = dis
    u1_ref[...] = z * dis[:, None]


def _tc2_body(dis_ref, u1_ref, s1_ref, b1_ref, w2p_ref, u2_ref):
    dis = dis_ref[...]
    ssum = s1_ref[0, :N_NODES, :] + s1_ref[1, :N_NODES, :]
    y1 = (ssum + u1_ref[...]) * dis[:, None] + b1_ref[...][None, :]
    h = jnp.maximum(y1, 0.0)
    z2 = jnp.dot(h, w2p_ref[...], preferred_element_type=jnp.float32)
    u2_ref[...] = z2 * dis[:, None]


def _tc3_body(dis_ref, u2_ref, s2_ref, b2p_ref, out_ref):
    dis = dis_ref[...]
    ssum = s2_ref[0, :N_NODES, :] + s2_ref[1, :N_NODES, :]
    y = (ssum + u2_ref[...]) * dis[:, None] + b2p_ref[...][None, :]
    col = lax.broadcasted_iota(jnp.int32, (N_NODES, F), 1)
    y = jnp.where(col < N_CLS, y, -1e30)
    m = jnp.max(y, axis=1, keepdims=True)
    lse = jnp.log(jnp.sum(jnp.exp(y - m), axis=1, keepdims=True))
    ls = y - m - lse
    out_ref[...] = ls[:, :N_CLS]


_tc1 = pl.pallas_call(
    _tc1_body,
    out_shape=[
        jax.ShapeDtypeStruct((N_NODES,), jnp.float32),
        jax.ShapeDtypeStruct((N_NODES, F), jnp.float32),
    ],
)

_tc2 = pl.pallas_call(
    _tc2_body,
    out_shape=jax.ShapeDtypeStruct((N_NODES, F), jnp.float32),
)

_tc3 = pl.pallas_call(
    _tc3_body,
    out_shape=jax.ShapeDtypeStruct((N_NODES, N_CLS), jnp.float32),
)


def kernel(x, edge_index, W1, b1, W2, b2):
    src2 = edge_index[0].astype(jnp.int32).reshape(NCHUNKS, CHUNK)
    dst2 = edge_index[1].astype(jnp.int32).reshape(NCHUNKS, CHUNK)
    w2p = jnp.pad(W2, ((0, 0), (0, F - N_CLS)))
    b2p = jnp.pad(b2, (0, F - N_CLS))

    degp = _deg_kernel(dst2)
    dis, u1 = _tc1(x, W1, degp)
    s1 = _prop_kernel(u1, src2, dst2)
    u2 = _tc2(dis, u1, s1, b1, w2p)
    s2 = _prop_kernel(u2, src2, dst2)
    return _tc3(dis, u2, s2, b2p)
